# initial kernel scaffold (unmeasured)
import jax
import jax.numpy as jnp
from jax import lax
from jax.experimental import pallas as pl
from jax.experimental.pallas import tpu as pltpu

M = 4096
D = 4096
CH = 512
NC = M // CH


def kernel(partial, gamma):
    gamma2d = gamma.reshape(1, D)

    def body(partial_ref, gamma_ref, out_ref,
             send_hbm, recv_hbm,
             v_local, v_send, v_recv, v_out,
             local_sem, send_sems, recv_sems):
        my_x = lax.axis_index("x")
        my_y = lax.axis_index("y")
        my_z = lax.axis_index("z")
        nbr = (my_x, 1 - my_y, my_z)

        barrier = pltpu.get_barrier_semaphore()
        pl.semaphore_signal(barrier, inc=1, device_id=nbr,
                            device_id_type=pl.DeviceIdType.MESH)
        pl.semaphore_wait(barrier, 1)

        send_base = (1 - my_y) * M
        my_base = my_y * M

        rdmas = []
        for c in range(NC):
            cp = pltpu.make_async_copy(
                partial_ref.at[0, pl.ds(send_base + c * CH, CH), :],
                v_local, local_sem)
            cp.start()
            cp.wait()
            v_send[...] = v_local[...].astype(jnp.bfloat16)
            cp2 = pltpu.make_async_copy(
                v_send, send_hbm.at[pl.ds(c * CH, CH), :], local_sem)
            cp2.start()
            cp2.wait()
            rdma = pltpu.make_async_remote_copy(
                src_ref=send_hbm.at[pl.ds(c * CH, CH), :],
                dst_ref=recv_hbm.at[pl.ds(c * CH, CH), :],
                send_sem=send_sems.at[c],
                recv_sem=recv_sems.at[c],
                device_id=nbr,
                device_id_type=pl.DeviceIdType.MESH,
            )
            rdma.start()
            rdmas.append(rdma)

        for c in range(NC):
            rdmas[c].wait_recv()
            cpl = pltpu.make_async_copy(
                partial_ref.at[0, pl.ds(my_base + c * CH, CH), :],
                v_local, local_sem)
            cpl.start()
            cpl.wait()
            cpr = pltpu.make_async_copy(
                recv_hbm.at[pl.ds(c * CH, CH), :], v_recv, local_sem)
            cpr.start()
            cpr.wait()
            y = v_local[...] + v_recv[...].astype(jnp.float32)
            ms = jnp.mean(y * y, axis=1, keepdims=True)
            v_out[...] = y * lax.rsqrt(ms + 1e-6) * gamma_ref[...]
            cpo = pltpu.make_async_copy(
                v_out, out_ref.at[pl.ds(c * CH, CH), :], local_sem)
            cpo.start()
            cpo.wait()

        for c in range(NC):
            rdmas[c].wait_send()

    out_shape = jax.ShapeDtypeStruct((M, D), jnp.float32)
    return pl.pallas_call(
        body,
        out_shape=out_shape,
        in_specs=[
            pl.BlockSpec(memory_space=pl.ANY),
            pl.BlockSpec(memory_space=pltpu.MemorySpace.VMEM),
        ],
        out_specs=pl.BlockSpec(memory_space=pl.ANY),
        scratch_shapes=[
            pltpu.MemorySpace.HBM((M, D), jnp.bfloat16),
            pltpu.MemorySpace.HBM((M, D), jnp.bfloat16),
            pltpu.VMEM((CH, D), jnp.float32),
            pltpu.VMEM((CH, D), jnp.bfloat16),
            pltpu.VMEM((CH, D), jnp.bfloat16),
            pltpu.VMEM((CH, D), jnp.float32),
            pltpu.SemaphoreType.DMA,
            pltpu.SemaphoreType.DMA((NC,)),
            pltpu.SemaphoreType.DMA((NC,)),
        ],
        compiler_params=pltpu.CompilerParams(collective_id=0),
    )(partial, gamma2d)


# baseline (device time: 420505 ns/iter reference)
import jax
import jax.numpy as jnp
from jax import lax
from jax.experimental import pallas as pl
from jax.experimental.pallas import tpu as pltpu

M = 4096
D = 4096
CH = 256
NC = M // CH


def kernel(partial, gamma):
    gamma2d = gamma.reshape(1, D)

    def body(partial_ref, gamma_ref, out_ref, send_hbm, recv_hbm,
             v_local, v_send, v_recv, v_out,
             local_sem, send_sems, recv_sems):
        my_x = lax.axis_index("x")
        my_y = lax.axis_index("y")
        my_z = lax.axis_index("z")
        nbr = (my_x, 1 - my_y, my_z)

        barrier = pltpu.get_barrier_semaphore()
        pl.semaphore_signal(barrier, inc=1, device_id=nbr,
                            device_id_type=pl.DeviceIdType.MESH)
        pl.semaphore_wait(barrier, 1)

        send_base = (1 - my_y) * M
        my_base = my_y * M

        rdmas = []
        for c in range(NC):
            cp = pltpu.make_async_copy(
                partial_ref.at[0, pl.ds(send_base + c * CH, CH), :],
                v_local, local_sem)
            cp.start()
            cp.wait()
            v_send[...] = v_local[...].astype(jnp.bfloat16)
            cp2 = pltpu.make_async_copy(
                v_send, send_hbm.at[pl.ds(c * CH, CH), :], local_sem)
            cp2.start()
            cp2.wait()
            rdma = pltpu.make_async_remote_copy(
                src_ref=send_hbm.at[pl.ds(c * CH, CH), :],
                dst_ref=recv_hbm.at[pl.ds(c * CH, CH), :],
                send_sem=send_sems.at[c],
                recv_sem=recv_sems.at[c],
                device_id=nbr,
                device_id_type=pl.DeviceIdType.MESH,
            )
            rdma.start()
            rdmas.append(rdma)

        for c in range(NC):
            rdmas[c].wait_recv()
            cpl = pltpu.make_async_copy(
                partial_ref.at[0, pl.ds(my_base + c * CH, CH), :],
                v_local, local_sem)
            cpl.start()
            cpl.wait()
            cpr = pltpu.make_async_copy(
                recv_hbm.at[pl.ds(c * CH, CH), :], v_recv, local_sem)
            cpr.start()
            cpr.wait()
            y = v_local[...] + v_recv[...].astype(jnp.float32)
            ms = jnp.mean(y * y, axis=1, keepdims=True)
            v_out[...] = y * lax.rsqrt(ms + 1e-6) * gamma_ref[...]
            cpo = pltpu.make_async_copy(
                v_out, out_ref.at[pl.ds(c * CH, CH), :], local_sem)
            cpo.start()
            cpo.wait()

        for c in range(NC):
            rdmas[c].wait_send()

    out_shape = [
        jax.ShapeDtypeStruct((M, D), jnp.float32),
        jax.ShapeDtypeStruct((M, D), jnp.bfloat16),
        jax.ShapeDtypeStruct((M, D), jnp.bfloat16),
    ]
    outs = pl.pallas_call(
        body,
        out_shape=out_shape,
        in_specs=[
            pl.BlockSpec(memory_space=pl.ANY),
            pl.BlockSpec(memory_space=pltpu.MemorySpace.VMEM),
        ],
        out_specs=[
            pl.BlockSpec(memory_space=pl.ANY),
            pl.BlockSpec(memory_space=pl.ANY),
            pl.BlockSpec(memory_space=pl.ANY),
        ],
        scratch_shapes=[
            pltpu.VMEM((CH, D), jnp.float32),
            pltpu.VMEM((CH, D), jnp.bfloat16),
            pltpu.VMEM((CH, D), jnp.bfloat16),
            pltpu.VMEM((CH, D), jnp.float32),
            pltpu.SemaphoreType.DMA,
            pltpu.SemaphoreType.DMA((NC,)),
            pltpu.SemaphoreType.DMA((NC,)),
        ],
        compiler_params=pltpu.CompilerParams(collective_id=0),
    )(partial, gamma2d)
    return outs[0]


# device time: 345310 ns/iter; 1.2178x vs baseline; 1.2178x over previous
import jax
import jax.numpy as jnp
from jax import lax
from jax.experimental import pallas as pl
from jax.experimental.pallas import tpu as pltpu

M = 4096
D = 4096
QR = 1024
CH = 256
NY = 8
NF = 4


def kernel(partial, gamma):
    gamma2d = gamma.reshape(1, D)

    def body(partial_ref, gamma_ref, out_ref, send_hbm, recv_hbm,
             v_local, v_send, v_recv, v_out, local_sem,
             y_send_sems, y_recv_sems,
             x_send_sems, x_recv_sems,
             z_send_sems, z_recv_sems):
        my_x = lax.axis_index("x")
        my_y = lax.axis_index("y")
        my_z = lax.axis_index("z")
        y_nbr = (my_x, 1 - my_y, my_z)
        x_nbr = (1 - my_x, my_y, my_z)
        z_nbr = (my_x, my_y, 1 - my_z)

        barrier = pltpu.get_barrier_semaphore()
        for nbr in (y_nbr, x_nbr, z_nbr):
            pl.semaphore_signal(barrier, inc=1, device_id=nbr,
                                device_id_type=pl.DeviceIdType.MESH)
        pl.semaphore_wait(barrier, 3)

        pair = jnp.where(my_x == my_z, 0, 1)
        y_base = pair * 2 * QR
        x_base = (1 - pair) * 2 * QR
        z_base = x_base + QR
        send_base = (1 - my_y) * M
        my_base = my_y * M

        def block_off(k):
            return y_base + (k % 2) * QR + (k // 2) * CH

        y_rdmas = []
        for k in range(NY):
            off = block_off(k)
            cp = pltpu.make_async_copy(
                partial_ref.at[0, pl.ds(send_base + off, CH), :],
                v_local, local_sem)
            cp.start()
            cp.wait()
            v_send[...] = v_local[...].astype(jnp.bfloat16)
            cp2 = pltpu.make_async_copy(
                v_send, send_hbm.at[pl.ds(k * CH, CH), :], local_sem)
            cp2.start()
            cp2.wait()
            rdma = pltpu.make_async_remote_copy(
                src_ref=send_hbm.at[pl.ds(k * CH, CH), :],
                dst_ref=recv_hbm.at[pl.ds(off, CH), :],
                send_sem=y_send_sems.at[k],
                recv_sem=y_recv_sems.at[k],
                device_id=y_nbr,
                device_id_type=pl.DeviceIdType.MESH,
            )
            rdma.start()
            y_rdmas.append(rdma)

        fwd_rdmas = []
        for k in range(NY):
            y_rdmas[k].wait_recv()
            off = block_off(k)
            j = k // 2
            if k % 2 == 0:
                tgt, ss, rs = x_nbr, x_send_sems, x_recv_sems
            else:
                tgt, ss, rs = z_nbr, z_send_sems, z_recv_sems
            r = pltpu.make_async_remote_copy(
                src_ref=recv_hbm.at[pl.ds(off, CH), :],
                dst_ref=recv_hbm.at[pl.ds(off, CH), :],
                send_sem=ss.at[j],
                recv_sem=rs.at[j],
                device_id=tgt,
                device_id_type=pl.DeviceIdType.MESH,
            )
            r.start()
            fwd_rdmas.append(r)

        def fwd_recv(base, j, ss, rs, tgt):
            return pltpu.make_async_remote_copy(
                src_ref=recv_hbm.at[pl.ds(base + j * CH, CH), :],
                dst_ref=recv_hbm.at[pl.ds(base + j * CH, CH), :],
                send_sem=ss.at[j],
                recv_sem=rs.at[j],
                device_id=tgt,
                device_id_type=pl.DeviceIdType.MESH,
            )

        order = [("y", 0), ("y", 1), ("x", 0), ("y", 2), ("z", 0), ("y", 3),
                 ("x", 1), ("y", 4), ("z", 1), ("y", 5), ("x", 2), ("y", 6),
                 ("z", 2), ("y", 7), ("x", 3), ("z", 3)]
        for kind, i in order:
            if kind == "y":
                off = block_off(i)
            elif kind == "x":
                off = x_base + i * CH
                fwd_recv(x_base, i, x_send_sems, x_recv_sems, x_nbr).wait_recv()
            else:
                off = z_base + i * CH
                fwd_recv(z_base, i, z_send_sems, z_recv_sems, z_nbr).wait_recv()
            cpl = pltpu.make_async_copy(
                partial_ref.at[0, pl.ds(my_base + off, CH), :],
                v_local, local_sem)
            cpl.start()
            cpl.wait()
            cpr = pltpu.make_async_copy(
                recv_hbm.at[pl.ds(off, CH), :], v_recv, local_sem)
            cpr.start()
            cpr.wait()
            y = v_local[...] + v_recv[...].astype(jnp.float32)
            ms = jnp.mean(y * y, axis=1, keepdims=True)
            v_out[...] = y * lax.rsqrt(ms + 1e-6) * gamma_ref[...]
            cpo = pltpu.make_async_copy(
                v_out, out_ref.at[pl.ds(off, CH), :], local_sem)
            cpo.start()
            cpo.wait()

        for r in y_rdmas:
            r.wait_send()
        for r in fwd_rdmas:
            r.wait_send()

    out_shape = [
        jax.ShapeDtypeStruct((M, D), jnp.float32),
        jax.ShapeDtypeStruct((2 * QR, D), jnp.bfloat16),
        jax.ShapeDtypeStruct((M, D), jnp.bfloat16),
    ]
    outs = pl.pallas_call(
        body,
        out_shape=out_shape,
        in_specs=[
            pl.BlockSpec(memory_space=pl.ANY),
            pl.BlockSpec(memory_space=pltpu.MemorySpace.VMEM),
        ],
        out_specs=[
            pl.BlockSpec(memory_space=pl.ANY),
            pl.BlockSpec(memory_space=pl.ANY),
            pl.BlockSpec(memory_space=pl.ANY),
        ],
        scratch_shapes=[
            pltpu.VMEM((CH, D), jnp.float32),
            pltpu.VMEM((CH, D), jnp.bfloat16),
            pltpu.VMEM((CH, D), jnp.bfloat16),
            pltpu.VMEM((CH, D), jnp.float32),
            pltpu.SemaphoreType.DMA,
            pltpu.SemaphoreType.DMA((NY,)),
            pltpu.SemaphoreType.DMA((NY,)),
            pltpu.SemaphoreType.DMA((NF,)),
            pltpu.SemaphoreType.DMA((NF,)),
            pltpu.SemaphoreType.DMA((NF,)),
            pltpu.SemaphoreType.DMA((NF,)),
        ],
        compiler_params=pltpu.CompilerParams(collective_id=0),
    )(partial, gamma2d)
    return outs[0]


# device time: 291290 ns/iter; 1.4436x vs baseline; 1.1855x over previous
import os

import jax
import jax.numpy as jnp
from jax import lax
from jax.experimental import pallas as pl
from jax.experimental.pallas import tpu as pltpu

DEBUG = os.environ.get("KERNEL_DEBUG", "")

M = 4096
D = 4096
QR = 1024
CH = 256
NY = 8
NF = 4
NSB = 4
NT = NY + 2 * NF


def kernel(partial, gamma):
    gamma2d = gamma.reshape(1, D)

    def body(partial_ref, gamma_ref, out_ref, recv_hbm,
             v_stage, v_send, v_local, v_recv, v_out,
             stage_sems, in_sems, out_sems,
             y_send_sems, y_recv_sems,
             x_send_sems, x_recv_sems,
             z_send_sems, z_recv_sems):
        my_x = lax.axis_index("x")
        my_y = lax.axis_index("y")
        my_z = lax.axis_index("z")
        y_nbr = (my_x, 1 - my_y, my_z)
        x_nbr = (1 - my_x, my_y, my_z)
        z_nbr = (my_x, my_y, 1 - my_z)

        barrier = pltpu.get_barrier_semaphore()
        for nbr in (y_nbr, x_nbr, z_nbr):
            pl.semaphore_signal(barrier, inc=1, device_id=nbr,
                                device_id_type=pl.DeviceIdType.MESH)
        pl.semaphore_wait(barrier, 3)

        pair = jnp.where(my_x == my_z, 0, 1)
        y_base = pair * 2 * QR
        x_base = (1 - pair) * 2 * QR
        z_base = x_base + QR
        send_base = (1 - my_y) * M
        my_base = my_y * M

        def block_off(k):
            return y_base + (k % 2) * QR + (k // 2) * CH

        def stage(k):
            return pltpu.make_async_copy(
                partial_ref.at[0, pl.ds(send_base + block_off(k), CH), :],
                v_stage.at[k % 2], stage_sems.at[k % 2])

        y_rdmas = []
        stage(0).start()
        for k in range(NY):
            if k + 1 < NY:
                stage(k + 1).start()
            stage(k).wait()
            sb = k % NSB
            if k >= NSB:
                y_rdmas[k - NSB].wait_send()
            v_send[sb, :, :] = v_stage[k % 2].astype(jnp.bfloat16)
            rdma = pltpu.make_async_remote_copy(
                src_ref=v_send.at[sb],
                dst_ref=recv_hbm.at[pl.ds(block_off(k), CH), :],
                send_sem=y_send_sems.at[k],
                recv_sem=y_recv_sems.at[k],
                device_id=y_nbr,
                device_id_type=pl.DeviceIdType.MESH,
            )
            rdma.start()
            y_rdmas.append(rdma)

        fwd_rdmas = []
        for k in range(NY):
            y_rdmas[k].wait_recv()
            if DEBUG == "y_only":
                continue
            off = block_off(k)
            j = k // 2
            if k % 2 == 0:
                tgt, ss, rs = x_nbr, x_send_sems, x_recv_sems
            else:
                tgt, ss, rs = z_nbr, z_send_sems, z_recv_sems
            r = pltpu.make_async_remote_copy(
                src_ref=recv_hbm.at[pl.ds(off, CH), :],
                dst_ref=recv_hbm.at[pl.ds(off, CH), :],
                send_sem=ss.at[j],
                recv_sem=rs.at[j],
                device_id=tgt,
                device_id_type=pl.DeviceIdType.MESH,
            )
            r.start()
            fwd_rdmas.append(r)

        def fwd_recv(base, j, ss, rs, tgt):
            return pltpu.make_async_remote_copy(
                src_ref=recv_hbm.at[pl.ds(base + j * CH, CH), :],
                dst_ref=recv_hbm.at[pl.ds(base + j * CH, CH), :],
                send_sem=ss.at[j],
                recv_sem=rs.at[j],
                device_id=tgt,
                device_id_type=pl.DeviceIdType.MESH,
            )

        if DEBUG:
            if DEBUG == "comm_only":
                for j in range(NF):
                    fwd_recv(x_base, j, x_send_sems, x_recv_sems, x_nbr).wait_recv()
                    fwd_recv(z_base, j, z_send_sems, z_recv_sems, z_nbr).wait_recv()
            for r in y_rdmas[NY - NSB:]:
                r.wait_send()
            for r in fwd_rdmas:
                r.wait_send()
            return

        order = [("y", 0), ("y", 1), ("x", 0), ("y", 2), ("z", 0), ("y", 3),
                 ("x", 1), ("y", 4), ("z", 1), ("y", 5), ("x", 2), ("y", 6),
                 ("z", 2), ("y", 7), ("x", 3), ("z", 3)]

        def off_of(kind, i):
            if kind == "y":
                return block_off(i)
            return (x_base if kind == "x" else z_base) + i * CH

        def arrival_wait(kind, i):
            if kind == "x":
                fwd_recv(x_base, i, x_send_sems, x_recv_sems, x_nbr).wait_recv()
            elif kind == "z":
                fwd_recv(z_base, i, z_send_sems, z_recv_sems, z_nbr).wait_recv()

        def start_in(idx):
            kind, i = order[idx]
            s = idx % 2
            off = off_of(kind, i)
            cpl = pltpu.make_async_copy(
                partial_ref.at[0, pl.ds(my_base + off, CH), :],
                v_local.at[s], in_sems.at[2 * s])
            cpr = pltpu.make_async_copy(
                recv_hbm.at[pl.ds(off, CH), :], v_recv.at[s],
                in_sems.at[2 * s + 1])
            cpl.start()
            cpr.start()
            return cpl, cpr

        arrival_wait(*order[0])
        pend = {0: start_in(0)}
        outcps = {}
        for idx in range(NT):
            if idx + 1 < NT:
                arrival_wait(*order[idx + 1])
                pend[idx + 1] = start_in(idx + 1)
            cpl, cpr = pend.pop(idx)
            cpl.wait()
            cpr.wait()
            s = idx % 2
            if idx >= 2:
                outcps[idx - 2].wait()
            y = v_local[s] + v_recv[s].astype(jnp.float32)
            ms = jnp.mean(y * y, axis=1, keepdims=True)
            v_out[s, :, :] = y * lax.rsqrt(ms + 1e-6) * gamma_ref[...]
            kind, i = order[idx]
            oc = pltpu.make_async_copy(
                v_out.at[s], out_ref.at[pl.ds(off_of(kind, i), CH), :],
                out_sems.at[s])
            oc.start()
            outcps[idx] = oc

        outcps[NT - 2].wait()
        outcps[NT - 1].wait()
        for r in y_rdmas[NY - NSB:]:
            r.wait_send()
        for r in fwd_rdmas:
            r.wait_send()

    out_shape = [
        jax.ShapeDtypeStruct((M, D), jnp.float32),
        jax.ShapeDtypeStruct((M, D), jnp.bfloat16),
    ]
    outs = pl.pallas_call(
        body,
        out_shape=out_shape,
        in_specs=[
            pl.BlockSpec(memory_space=pl.ANY),
            pl.BlockSpec(memory_space=pltpu.MemorySpace.VMEM),
        ],
        out_specs=[
            pl.BlockSpec(memory_space=pl.ANY),
            pl.BlockSpec(memory_space=pl.ANY),
        ],
        scratch_shapes=[
            pltpu.VMEM((2, CH, D), jnp.float32),
            pltpu.VMEM((NSB, CH, D), jnp.bfloat16),
            pltpu.VMEM((2, CH, D), jnp.float32),
            pltpu.VMEM((2, CH, D), jnp.bfloat16),
            pltpu.VMEM((2, CH, D), jnp.float32),
            pltpu.SemaphoreType.DMA((2,)),
            pltpu.SemaphoreType.DMA((4,)),
            pltpu.SemaphoreType.DMA((2,)),
            pltpu.SemaphoreType.DMA((NY,)),
            pltpu.SemaphoreType.DMA((NY,)),
            pltpu.SemaphoreType.DMA((NF,)),
            pltpu.SemaphoreType.DMA((NF,)),
            pltpu.SemaphoreType.DMA((NF,)),
            pltpu.SemaphoreType.DMA((NF,)),
        ],
        compiler_params=pltpu.CompilerParams(
            collective_id=0,
            vmem_limit_bytes=48 * 1024 * 1024,
        ),
    )(partial, gamma2d)
    return outs[0]
